# Initial kernel scaffold; baseline (speedup 1.0000x reference)
import jax
import jax.numpy as jnp
from jax.experimental import pallas as pl


def kernel(x, emb, W0, b0, W1, b1, W2, b2, W3, b3, W4, b4):
    raise NotImplementedError("write your pallas kernel here")


# speedup vs baseline: 1.1882x; 1.1882x over previous
import jax
import jax.numpy as jnp
from jax import lax
from jax.experimental import pallas as pl
from jax.experimental.pallas import tpu as pltpu
from jax.experimental.pallas import tpu_sc as plsc

VOCAB = 1000
EMBED_DIM = 64
OUT_DIM = 1000
PAD_DIM = 1024
BATCH = 16384

_NC = 2
_NS = 16
_NW = _NC * _NS
_BPW = BATCH // _NW
_CHUNK = 32
_NCH = _BPW // _CHUNK


def _mlp_table_body(emb_ref, w0, b0, w1, b1, w2, b2, w3, b3, w4, b4, out_ref):
    h = jnp.maximum(emb_ref[...], 0.0)
    for w, b in ((w0, b0), (w1, b1), (w2, b2), (w3, b3), (w4, b4)):
        h = jnp.dot(h, w[...], preferred_element_type=jnp.float32) + b[...]
        h = jnp.maximum(h, 0.0)
    out_ref[...] = h


def _mlp_table(emb, ws, bs):
    args = [emb]
    for w, b in zip(ws, bs):
        args += [w, b.reshape(1, -1)]
    return pl.pallas_call(
        _mlp_table_body,
        out_shape=jax.ShapeDtypeStruct((VOCAB, OUT_DIM), jnp.float32),
    )(*args)


def _gather_body(x_ref, tab_ref, out_ref, idx_v, rows_v, sem0, sem1):
    wid = lax.axis_index("s") * _NC + lax.axis_index("c")
    base = wid * _BPW
    pltpu.sync_copy(x_ref.at[wid], idx_v)
    sems = (sem0, sem1)
    copies = [None, None]
    copies[0] = pltpu.async_copy(tab_ref.at[idx_v.at[0]], rows_v.at[0], sems[0])
    for ch in range(_NCH):
        buf = ch % 2
        if ch + 1 < _NCH:
            nb = (ch + 1) % 2
            copies[nb] = pltpu.async_copy(
                tab_ref.at[idx_v.at[ch + 1]], rows_v.at[nb], sems[nb])
        copies[buf].wait()
        pltpu.sync_copy(rows_v.at[buf],
                        out_ref.at[pl.ds(base + ch * _CHUNK, _CHUNK)])


def _gather(x3, table):
    return pl.kernel(
        _gather_body,
        out_type=jax.ShapeDtypeStruct((BATCH, OUT_DIM), jnp.float32),
        mesh=plsc.VectorSubcoreMesh(core_axis_name="c", subcore_axis_name="s"),
        compiler_params=pltpu.CompilerParams(use_tc_tiling_on_sc=False),
        scratch_types=[
            pltpu.VMEM((_NCH, _CHUNK), jnp.int32),
            pltpu.VMEM((2, _CHUNK, OUT_DIM), jnp.float32),
            pltpu.SemaphoreType.DMA,
            pltpu.SemaphoreType.DMA,
        ],
    )(x3, table)


def kernel(x, emb, W0, b0, W1, b1, W2, b2, W3, b3, W4, b4):
    table = _mlp_table(emb, (W0, W1, W2, W3, W4), (b0, b1, b2, b3, b4))
    x3 = x.reshape(_NW, _NCH, _CHUNK)
    return _gather(x3, table)


# speedup vs baseline: 1.8208x; 1.5325x over previous
import jax
import jax.numpy as jnp
from jax import lax
from jax.experimental import pallas as pl
from jax.experimental.pallas import tpu as pltpu
from jax.experimental.pallas import tpu_sc as plsc

VOCAB = 1000
EMBED_DIM = 64
OUT_DIM = 1000
PAD_DIM = 1024
BATCH = 16384

_NC = 2
_NS = 16
_NW = _NC * _NS
_BPW = BATCH // _NW
_CHUNK = 32
_NCH = _BPW // _CHUNK


def _mlp_table_body(emb_ref, w0, b0, w1, b1, w2, b2, w3, b3, w4, b4, out_ref):
    h = jnp.maximum(emb_ref[...], 0.0)
    for w, b in ((w0, b0), (w1, b1), (w2, b2), (w3, b3), (w4, b4)):
        h = jnp.dot(h, w[...], preferred_element_type=jnp.float32) + b[...]
        h = jnp.maximum(h, 0.0)
    out_ref[...] = jnp.concatenate(
        [h, jnp.zeros((VOCAB, PAD_DIM - OUT_DIM), jnp.float32)], axis=1)


def _mlp_table(emb, ws, bs):
    args = [emb]
    for w, b in zip(ws, bs):
        args += [w, b.reshape(1, -1)]
    return pl.pallas_call(
        _mlp_table_body,
        out_shape=jax.ShapeDtypeStruct((VOCAB, PAD_DIM), jnp.float32),
    )(*args)


def _gather_body(x_ref, tab_ref, out_ref, idx_v, rows_v, sem0, sem1):
    wid = lax.axis_index("s") * _NC + lax.axis_index("c")
    base = wid * _BPW
    pltpu.sync_copy(x_ref.at[wid], idx_v)
    sems = (sem0, sem1)
    copies = [None, None]
    copies[0] = pltpu.async_copy(tab_ref.at[idx_v.at[0]], rows_v.at[0], sems[0])
    for ch in range(_NCH):
        buf = ch % 2
        if ch + 1 < _NCH:
            nb = (ch + 1) % 2
            copies[nb] = pltpu.async_copy(
                tab_ref.at[idx_v.at[ch + 1]], rows_v.at[nb], sems[nb])
        copies[buf].wait()
        pltpu.sync_copy(rows_v.at[buf],
                        out_ref.at[pl.ds(base + ch * _CHUNK, _CHUNK)])


def _gather(x3, table):
    return pl.kernel(
        _gather_body,
        out_type=jax.ShapeDtypeStruct((BATCH, PAD_DIM), jnp.float32),
        mesh=plsc.VectorSubcoreMesh(core_axis_name="c", subcore_axis_name="s"),
        scratch_types=[
            pltpu.VMEM((_NCH, _CHUNK), jnp.int32),
            pltpu.VMEM((2, _CHUNK, PAD_DIM), jnp.float32),
            pltpu.SemaphoreType.DMA,
            pltpu.SemaphoreType.DMA,
        ],
    )(x3, table)


def kernel(x, emb, W0, b0, W1, b1, W2, b2, W3, b3, W4, b4):
    table = _mlp_table(emb, (W0, W1, W2, W3, W4), (b0, b1, b2, b3, b4))
    x3 = x.reshape(_NW, _NCH, _CHUNK)
    padded = _gather(x3, table)
    return lax.slice(padded, (0, 0), (BATCH, OUT_DIM))
